# baseline (speedup 1.0000x reference)
import jax
import jax.numpy as jnp
from jax.experimental import pallas as pl
from jax.experimental.pallas import tpu as pltpu

IN_F = 4096
OUT_F = 4096
BATCH = 64
K_BLK = 256
NCHUNK = IN_F // K_BLK
NBUF = 6


def _mm_kernel(x_ref, w_hbm, o_ref, bufs, xt_ref, acc_ref, sems):
    def copy(c):
        slot = c % NBUF
        return pltpu.make_async_copy(
            w_hbm.at[pl.ds(c * K_BLK, K_BLK), :],
            bufs.at[slot],
            sems.at[slot],
        )

    for c in range(NBUF - 1):
        copy(c).start()

    xt_ref[...] = x_ref[...].T.astype(jnp.bfloat16)

    def dot_chunk(c):
        copy(c).wait()
        if c + NBUF - 1 < NCHUNK:
            copy(c + NBUF - 1).start()
        return jax.lax.dot_general(
            xt_ref[:, pl.ds(c * K_BLK, K_BLK)],
            bufs[c % NBUF].astype(jnp.bfloat16),
            dimension_numbers=(((1,), (0,)), ((), ())),
            preferred_element_type=jnp.float32,
        )

    for g in range(NCHUNK // 2):
        part = dot_chunk(2 * g) + dot_chunk(2 * g + 1)
        if g == 0:
            acc_ref[...] = part
        else:
            acc_ref[...] += part

    o_ref[...] = acc_ref[...].T


def kernel(in_values, weights):
    return pl.pallas_call(
        _mm_kernel,
        in_specs=[
            pl.BlockSpec((IN_F, BATCH), lambda: (0, 0)),
            pl.BlockSpec(memory_space=pl.ANY),
        ],
        out_specs=pl.BlockSpec((OUT_F, BATCH), lambda: (0, 0)),
        out_shape=jax.ShapeDtypeStruct((OUT_F, BATCH), jnp.float32),
        scratch_shapes=[
            pltpu.VMEM((NBUF, K_BLK, OUT_F), jnp.float32),
            pltpu.VMEM((BATCH, IN_F), jnp.bfloat16),
            pltpu.VMEM((BATCH, OUT_F), jnp.float32),
            pltpu.SemaphoreType.DMA((NBUF,)),
        ],
    )(in_values, weights)
